# Initial kernel scaffold; baseline (speedup 1.0000x reference)
import jax
import jax.numpy as jnp
from jax.experimental import pallas as pl


def kernel(node_emb, edge_attr, params, edge_index):
    raise NotImplementedError("write your pallas kernel here")


# speedup vs baseline: 8.7928x; 8.7928x over previous
import functools

import jax
import jax.numpy as jnp
from jax import lax
from jax.experimental import pallas as pl
from jax.experimental.pallas import tpu as pltpu
from jax.experimental.pallas import tpu_sc as plsc

N = 10000
E = 320000
D = 128

NCORE = 2
NSUB = 16
BLK = 64
NPAD = 10240
LANES = 16

NBLK = 158
EPT = NBLK * BLK
EP = NCORE * NSUB * EPT


def _dense_proj_kernel(x_ref, wl_ref, bl_ref, wr_ref, br_ref, xl_ref, xr_ref):
    x = x_ref[...]
    xl_ref[...] = jnp.dot(x, wl_ref[...], preferred_element_type=jnp.float32) + bl_ref[...]
    xr_ref[...] = jnp.dot(x, wr_ref[...], preferred_element_type=jnp.float32) + br_ref[...]


def _dense_proj(x, wl, bl, wr, br):
    hc = wl.shape[1]
    return pl.pallas_call(
        _dense_proj_kernel,
        compiler_params=pltpu.CompilerParams(vmem_limit_bytes=100 * 1024 * 1024),
        out_shape=(
            jax.ShapeDtypeStruct((N, hc), jnp.float32),
            jax.ShapeDtypeStruct((N, hc), jnp.float32),
        ),
    )(x, wl, bl.reshape(1, hc), wr, br.reshape(1, hc))


def _dense_proj3_kernel(x_ref, wl_ref, bl_ref, wr_ref, br_ref, wres_ref, bres_ref,
                        xl_ref, xr_ref, xres_ref):
    x = x_ref[...]
    xl_ref[...] = jnp.dot(x, wl_ref[...], preferred_element_type=jnp.float32) + bl_ref[...]
    xr_ref[...] = jnp.dot(x, wr_ref[...], preferred_element_type=jnp.float32) + br_ref[...]
    xres_ref[...] = jnp.dot(x, wres_ref[...], preferred_element_type=jnp.float32) + bres_ref[...]


def _dense_proj3(x, wl, bl, wr, br, wres, bres):
    hc = wl.shape[1]
    co = wres.shape[1]
    return pl.pallas_call(
        _dense_proj3_kernel,
        compiler_params=pltpu.CompilerParams(vmem_limit_bytes=100 * 1024 * 1024),
        out_shape=(
            jax.ShapeDtypeStruct((N, hc), jnp.float32),
            jax.ShapeDtypeStruct((N, hc), jnp.float32),
            jax.ShapeDtypeStruct((N, co), jnp.float32),
        ),
    )(x, wl, bl.reshape(1, hc), wr, br.reshape(1, hc), wres, bres.reshape(1, co))


def _edge_body(h, c, w_row, src_hbm, dst_hbm, ea_hbm, xl_hbm, xr_hbm, wab_hbm,
               z_hbm, out_hbm, src_v, dst_v, ea_v, xl_b, xr_b, sval, wab_v,
               acc, sem0, sem1):
    hc = h * c
    cid = lax.axis_index("c")
    sid = lax.axis_index("s")

    pltpu.sync_copy(wab_hbm, wab_v)

    rows_per_tile = NPAD // NSUB
    zbase = sid * rows_per_tile
    pltpu.sync_copy(z_hbm.at[pl.ds(zbase, rows_per_tile)],
                    acc.at[pl.ds(zbase, rows_per_tile)])
    plsc.subcore_barrier()

    ebase = (cid * NSUB + sid) * EPT
    lane_iota = lax.iota(jnp.int32, LANES)

    def block(blk, _):
        base = pl.multiple_of(ebase + blk * BLK, BLK)
        pltpu.sync_copy(src_hbm.at[pl.ds(base, BLK)], src_v)
        pltpu.sync_copy(dst_hbm.at[pl.ds(base, BLK)], dst_v)
        pltpu.sync_copy(ea_hbm.at[pl.ds(base, BLK)], ea_v)
        g0 = pltpu.async_copy(xl_hbm.at[src_v], xl_b, sem0)
        g1 = pltpu.async_copy(xr_hbm.at[dst_v], xr_b, sem1)
        g0.wait()
        g1.wait()

        def subblock(sb, _):
            rows = sb * LANES + lane_iota
            eav = ea_v[pl.ds(sb * LANES, LANES)]
            exs = []
            for hh in range(h):
                def jbody(jj, a):
                    j = hh * c + jj
                    cj = jnp.full((LANES,), j, jnp.int32)
                    xlv = plsc.load_gather(xl_b, [rows, cj])
                    xrv = plsc.load_gather(xr_b, [rows, cj])
                    wspl = wab_v[0, j, :]
                    aspl = wab_v[1, j, :]
                    t = xlv + xrv + eav * wspl
                    t = jnp.maximum(t, 0.2 * t)
                    return a + t * aspl
                alpha = lax.fori_loop(0, c, jbody, jnp.zeros((LANES,), jnp.float32))
                exs.append(jnp.exp(alpha))
            for hh in range(h):
                exv = exs[hh]
                def j2(jj, c2):
                    j = hh * c + jj
                    cj = jnp.full((LANES,), j, jnp.int32)
                    xlv = plsc.load_gather(xl_b, [rows, cj])
                    plsc.store_scatter(sval, [rows, cj], xlv * exv)
                    return c2
                lax.fori_loop(0, c, j2, None)
                ch = jnp.full((LANES,), hc + hh, jnp.int32)
                plsc.store_scatter(sval, [rows, ch], exv)
            return _

        lax.fori_loop(0, BLK // LANES, subblock, None)
        pltpu.sync_copy(sval, acc.at[dst_v], add=True)
        return _

    lax.fori_loop(0, NBLK, block, None)
    plsc.subcore_barrier()
    pltpu.sync_copy(acc.at[pl.ds(zbase, rows_per_tile)],
                    out_hbm.at[pl.ds(cid * NPAD + zbase, rows_per_tile)])


def _edge_phase(h, c, w_row, srcp, dstp, eap, xl, xr, wab, zrows):
    hc = h * c
    mesh = plsc.VectorSubcoreMesh(core_axis_name="c", subcore_axis_name="s",
                                  num_cores=NCORE, num_subcores=NSUB)
    body = functools.partial(_edge_body, h, c, w_row)
    f = pl.kernel(
        body,
        out_type=jax.ShapeDtypeStruct((NCORE * NPAD, w_row), jnp.float32),
        mesh=mesh,
        compiler_params=pltpu.CompilerParams(needs_layout_passes=False,
                                             use_tc_tiling_on_sc=False),
        scratch_types=[
            pltpu.VMEM((BLK,), jnp.int32),
            pltpu.VMEM((BLK,), jnp.int32),
            pltpu.VMEM((BLK,), jnp.float32),
            pltpu.VMEM((BLK, hc), jnp.float32),
            pltpu.VMEM((BLK, hc), jnp.float32),
            pltpu.VMEM((BLK, w_row), jnp.float32),
            pltpu.VMEM((2, hc, LANES), jnp.float32),
            pltpu.VMEM_SHARED((NPAD, w_row), jnp.float32),
            pltpu.SemaphoreType.DMA,
            pltpu.SemaphoreType.DMA,
        ],
    )
    return f(srcp, dstp, eap, xl, xr, wab, zrows)


def _post_kernel(h, c, part_ref, bias_ref, g_ref, b_ref, xres_ref, o_ref):
    hc = h * c
    p = part_ref[0:N, :] + part_ref[NPAD:NPAD + N, :]
    cols = []
    for hh in range(h):
        num = p[:, hh * c:(hh + 1) * c]
        den = p[:, hc + hh:hc + hh + 1]
        cols.append(num / (den + 1e-16))
    y = cols[0] if h == 1 else jnp.concatenate(cols, axis=1)
    y = y + bias_ref[...]
    y = jnp.where(y > 0, y, jnp.exp(y) - 1.0)
    mu = jnp.mean(y, axis=-1, keepdims=True)
    var = jnp.mean((y - mu) ** 2, axis=-1, keepdims=True)
    y = (y - mu) / jnp.sqrt(var + 1e-5) * g_ref[...] + b_ref[...]
    o_ref[...] = y + xres_ref[...]


def _post(h, c, part, bias, ln_g, ln_b, xres):
    hc = h * c
    co = hc if h > 1 else c
    kfn = functools.partial(_post_kernel, h, c)
    return pl.pallas_call(
        kfn,
        compiler_params=pltpu.CompilerParams(vmem_limit_bytes=100 * 1024 * 1024),
        out_shape=jax.ShapeDtypeStruct((N, co), jnp.float32),
    )(part, bias.reshape(1, co), ln_g.reshape(1, co), ln_b.reshape(1, co), xres)


def kernel(node_emb, edge_attr, params, edge_index):
    src = edge_index[0].astype(jnp.int32)
    dst = edge_index[1].astype(jnp.int32)
    ea = edge_attr[:, 0].astype(jnp.float32)
    pad = EP - E
    srcp = jnp.concatenate([src, jnp.zeros((pad,), jnp.int32)])
    dstp = jnp.concatenate([dst, jnp.full((pad,), N, jnp.int32)])
    eap = jnp.concatenate([ea, jnp.zeros((pad,), jnp.float32)])

    x = node_emb
    cfgs = [(4, 32, 144), (4, 32, 144), (1, 32, 48)]
    for i, (h, c, w_row) in enumerate(cfgs):
        hc = h * c
        P = params
        if i == 2:
            xl, xr, xres = _dense_proj3(x, P['Wl%d' % i], P['bl%d' % i],
                                        P['Wr%d' % i], P['br%d' % i],
                                        P['Wres2'], P['bres2'])
        else:
            xl, xr = _dense_proj(x, P['Wl%d' % i], P['bl%d' % i],
                                 P['Wr%d' % i], P['br%d' % i])
            xres = x
        wa = jnp.stack([P['We%d' % i].reshape(hc), P['att%d' % i].reshape(hc)])
        wab = jnp.broadcast_to(wa[:, :, None], (2, hc, LANES)).astype(jnp.float32)
        zrows = jnp.zeros((NPAD, w_row), jnp.float32)
        part = _edge_phase(h, c, w_row, srcp, dstp, eap, xl, xr, wab, zrows)
        x = _post(h, c, part, P['bias%d' % i], P['ln_g%d' % i], P['ln_b%d' % i], xres)
    return x


# speedup vs baseline: 8.8651x; 1.0082x over previous
import functools

import jax
import jax.numpy as jnp
from jax import lax
from jax.experimental import pallas as pl
from jax.experimental.pallas import tpu as pltpu
from jax.experimental.pallas import tpu_sc as plsc

N = 10000
E = 320000
D = 128

NCORE = 2
NSUB = 16
BLK = 64
NPAD = 10240
LANES = 16

NBLK = 158
EPT = NBLK * BLK
EP = NCORE * NSUB * EPT


def _dense_proj_kernel(x_ref, wl_ref, bl_ref, wr_ref, br_ref, xl_ref, xr_ref):
    x = x_ref[...]
    xl_ref[...] = jnp.dot(x, wl_ref[...], preferred_element_type=jnp.float32) + bl_ref[...]
    xr_ref[...] = jnp.dot(x, wr_ref[...], preferred_element_type=jnp.float32) + br_ref[...]


def _dense_proj(x, wl, bl, wr, br):
    hc = wl.shape[1]
    return pl.pallas_call(
        _dense_proj_kernel,
        compiler_params=pltpu.CompilerParams(vmem_limit_bytes=100 * 1024 * 1024),
        out_shape=(
            jax.ShapeDtypeStruct((N, hc), jnp.float32),
            jax.ShapeDtypeStruct((N, hc), jnp.float32),
        ),
    )(x, wl, bl.reshape(1, hc), wr, br.reshape(1, hc))


def _dense_proj3_kernel(x_ref, wl_ref, bl_ref, wr_ref, br_ref, wres_ref, bres_ref,
                        xl_ref, xr_ref, xres_ref):
    x = x_ref[...]
    xl_ref[...] = jnp.dot(x, wl_ref[...], preferred_element_type=jnp.float32) + bl_ref[...]
    xr_ref[...] = jnp.dot(x, wr_ref[...], preferred_element_type=jnp.float32) + br_ref[...]
    xres_ref[...] = jnp.dot(x, wres_ref[...], preferred_element_type=jnp.float32) + bres_ref[...]


def _dense_proj3(x, wl, bl, wr, br, wres, bres):
    hc = wl.shape[1]
    co = wres.shape[1]
    return pl.pallas_call(
        _dense_proj3_kernel,
        compiler_params=pltpu.CompilerParams(vmem_limit_bytes=100 * 1024 * 1024),
        out_shape=(
            jax.ShapeDtypeStruct((N, hc), jnp.float32),
            jax.ShapeDtypeStruct((N, hc), jnp.float32),
            jax.ShapeDtypeStruct((N, co), jnp.float32),
        ),
    )(x, wl, bl.reshape(1, hc), wr, br.reshape(1, hc), wres, bres.reshape(1, co))


def _edge_body(h, c, w_row, src_hbm, dst_hbm, ea_hbm, xl_hbm, xr_hbm, wab_hbm,
               z_hbm, out_hbm, src_v, dst_v, ea_v, xl_b, xr_b, sval, wab_v,
               acc, sem0, sem1):
    hc = h * c
    cid = lax.axis_index("c")
    sid = lax.axis_index("s")

    pltpu.sync_copy(wab_hbm, wab_v)

    rows_per_tile = NPAD // NSUB
    zbase = sid * rows_per_tile
    pltpu.sync_copy(z_hbm.at[pl.ds(zbase, rows_per_tile)],
                    acc.at[pl.ds(zbase, rows_per_tile)])
    plsc.subcore_barrier()

    ebase = (cid * NSUB + sid) * EPT
    lane_iota = lax.iota(jnp.int32, LANES)

    def block(blk, _):
        base = pl.multiple_of(ebase + blk * BLK, BLK)
        pltpu.sync_copy(src_hbm.at[pl.ds(base, BLK)], src_v)
        pltpu.sync_copy(dst_hbm.at[pl.ds(base, BLK)], dst_v)
        pltpu.sync_copy(ea_hbm.at[pl.ds(base, BLK)], ea_v)
        g0 = pltpu.async_copy(xl_hbm.at[src_v], xl_b, sem0)
        g1 = pltpu.async_copy(xr_hbm.at[dst_v], xr_b, sem1)
        g0.wait()
        g1.wait()

        def subblock(sb, _):
            rows = sb * LANES + lane_iota
            eav = ea_v[pl.ds(sb * LANES, LANES)]
            exs = []
            U = 8
            for hh in range(h):
                def jbody(jv, a):
                    for u in range(U):
                        j = hh * c + jv * U + u
                        cj = jnp.full((LANES,), j, jnp.int32)
                        xlv = plsc.load_gather(xl_b, [rows, cj])
                        xrv = plsc.load_gather(xr_b, [rows, cj])
                        wspl = wab_v[0, j, :]
                        aspl = wab_v[1, j, :]
                        t = xlv + xrv + eav * wspl
                        t = jnp.maximum(t, 0.2 * t)
                        a = a + t * aspl
                    return a
                alpha = lax.fori_loop(0, c // U, jbody,
                                      jnp.zeros((LANES,), jnp.float32))
                exs.append(jnp.exp(alpha))
            for hh in range(h):
                exv = exs[hh]
                def j2(jv, c2):
                    for u in range(U):
                        j = hh * c + jv * U + u
                        cj = jnp.full((LANES,), j, jnp.int32)
                        xlv = plsc.load_gather(xl_b, [rows, cj])
                        plsc.store_scatter(sval, [rows, cj], xlv * exv)
                    return c2
                lax.fori_loop(0, c // U, j2, None)
                ch = jnp.full((LANES,), hc + hh, jnp.int32)
                plsc.store_scatter(sval, [rows, ch], exv)
            return _

        lax.fori_loop(0, BLK // LANES, subblock, None)
        pltpu.sync_copy(sval, acc.at[dst_v], add=True)
        return _

    lax.fori_loop(0, NBLK, block, None)
    plsc.subcore_barrier()
    pltpu.sync_copy(acc.at[pl.ds(zbase, rows_per_tile)],
                    out_hbm.at[pl.ds(cid * NPAD + zbase, rows_per_tile)])


def _edge_phase(h, c, w_row, srcp, dstp, eap, xl, xr, wab, zrows):
    hc = h * c
    mesh = plsc.VectorSubcoreMesh(core_axis_name="c", subcore_axis_name="s",
                                  num_cores=NCORE, num_subcores=NSUB)
    body = functools.partial(_edge_body, h, c, w_row)
    f = pl.kernel(
        body,
        out_type=jax.ShapeDtypeStruct((NCORE * NPAD, w_row), jnp.float32),
        mesh=mesh,
        compiler_params=pltpu.CompilerParams(needs_layout_passes=False,
                                             use_tc_tiling_on_sc=False),
        scratch_types=[
            pltpu.VMEM((BLK,), jnp.int32),
            pltpu.VMEM((BLK,), jnp.int32),
            pltpu.VMEM((BLK,), jnp.float32),
            pltpu.VMEM((BLK, hc), jnp.float32),
            pltpu.VMEM((BLK, hc), jnp.float32),
            pltpu.VMEM((BLK, w_row), jnp.float32),
            pltpu.VMEM((2, hc, LANES), jnp.float32),
            pltpu.VMEM_SHARED((NPAD, w_row), jnp.float32),
            pltpu.SemaphoreType.DMA,
            pltpu.SemaphoreType.DMA,
        ],
    )
    return f(srcp, dstp, eap, xl, xr, wab, zrows)


def _post_kernel(h, c, part_ref, bias_ref, g_ref, b_ref, xres_ref, o_ref):
    hc = h * c
    p = part_ref[0:N, :] + part_ref[NPAD:NPAD + N, :]
    cols = []
    for hh in range(h):
        num = p[:, hh * c:(hh + 1) * c]
        den = p[:, hc + hh:hc + hh + 1]
        cols.append(num / (den + 1e-16))
    y = cols[0] if h == 1 else jnp.concatenate(cols, axis=1)
    y = y + bias_ref[...]
    y = jnp.where(y > 0, y, jnp.exp(y) - 1.0)
    mu = jnp.mean(y, axis=-1, keepdims=True)
    var = jnp.mean((y - mu) ** 2, axis=-1, keepdims=True)
    y = (y - mu) / jnp.sqrt(var + 1e-5) * g_ref[...] + b_ref[...]
    o_ref[...] = y + xres_ref[...]


def _post(h, c, part, bias, ln_g, ln_b, xres):
    hc = h * c
    co = hc if h > 1 else c
    kfn = functools.partial(_post_kernel, h, c)
    return pl.pallas_call(
        kfn,
        compiler_params=pltpu.CompilerParams(vmem_limit_bytes=100 * 1024 * 1024),
        out_shape=jax.ShapeDtypeStruct((N, co), jnp.float32),
    )(part, bias.reshape(1, co), ln_g.reshape(1, co), ln_b.reshape(1, co), xres)


def kernel(node_emb, edge_attr, params, edge_index):
    src = edge_index[0].astype(jnp.int32)
    dst = edge_index[1].astype(jnp.int32)
    ea = edge_attr[:, 0].astype(jnp.float32)
    pad = EP - E
    srcp = jnp.concatenate([src, jnp.zeros((pad,), jnp.int32)])
    dstp = jnp.concatenate([dst, jnp.full((pad,), N, jnp.int32)])
    eap = jnp.concatenate([ea, jnp.zeros((pad,), jnp.float32)])

    x = node_emb
    cfgs = [(4, 32, 144), (4, 32, 144), (1, 32, 48)]
    for i, (h, c, w_row) in enumerate(cfgs):
        hc = h * c
        P = params
        if i == 2:
            xl, xr, xres = _dense_proj3(x, P['Wl%d' % i], P['bl%d' % i],
                                        P['Wr%d' % i], P['br%d' % i],
                                        P['Wres2'], P['bres2'])
        else:
            xl, xr = _dense_proj(x, P['Wl%d' % i], P['bl%d' % i],
                                 P['Wr%d' % i], P['br%d' % i])
            xres = x
        wa = jnp.stack([P['We%d' % i].reshape(hc), P['att%d' % i].reshape(hc)])
        wab = jnp.broadcast_to(wa[:, :, None], (2, hc, LANES)).astype(jnp.float32)
        zrows = jnp.zeros((NPAD, w_row), jnp.float32)
        part = _edge_phase(h, c, w_row, srcp, dstp, eap, xl, xr, wab, zrows)
        x = _post(h, c, part, P['bias%d' % i], P['ln_g%d' % i], P['ln_b%d' % i], xres)
    return x


# speedup vs baseline: 8.9651x; 1.0113x over previous
import functools

import jax
import jax.numpy as jnp
from jax import lax
from jax.experimental import pallas as pl
from jax.experimental.pallas import tpu as pltpu
from jax.experimental.pallas import tpu_sc as plsc

N = 10000
E = 320000
D = 128

NCORE = 2
NSUB = 16
BLK = 64
NPAD = 10240
LANES = 16

NBLK = 160
G = 8
EPT = NBLK * BLK
EP = NCORE * NSUB * EPT


def _dense_proj_kernel(x_ref, wl_ref, bl_ref, wr_ref, br_ref, xl_ref, xr_ref):
    x = x_ref[...]
    xl_ref[...] = jnp.dot(x, wl_ref[...], preferred_element_type=jnp.float32) + bl_ref[...]
    xr_ref[...] = jnp.dot(x, wr_ref[...], preferred_element_type=jnp.float32) + br_ref[...]


def _dense_proj(x, wl, bl, wr, br):
    hc = wl.shape[1]
    return pl.pallas_call(
        _dense_proj_kernel,
        compiler_params=pltpu.CompilerParams(vmem_limit_bytes=100 * 1024 * 1024),
        out_shape=(
            jax.ShapeDtypeStruct((N, hc), jnp.float32),
            jax.ShapeDtypeStruct((N, hc), jnp.float32),
        ),
    )(x, wl, bl.reshape(1, hc), wr, br.reshape(1, hc))


def _dense_proj3_kernel(x_ref, wl_ref, bl_ref, wr_ref, br_ref, wres_ref, bres_ref,
                        xl_ref, xr_ref, xres_ref):
    x = x_ref[...]
    xl_ref[...] = jnp.dot(x, wl_ref[...], preferred_element_type=jnp.float32) + bl_ref[...]
    xr_ref[...] = jnp.dot(x, wr_ref[...], preferred_element_type=jnp.float32) + br_ref[...]
    xres_ref[...] = jnp.dot(x, wres_ref[...], preferred_element_type=jnp.float32) + bres_ref[...]


def _dense_proj3(x, wl, bl, wr, br, wres, bres):
    hc = wl.shape[1]
    co = wres.shape[1]
    return pl.pallas_call(
        _dense_proj3_kernel,
        compiler_params=pltpu.CompilerParams(vmem_limit_bytes=100 * 1024 * 1024),
        out_shape=(
            jax.ShapeDtypeStruct((N, hc), jnp.float32),
            jax.ShapeDtypeStruct((N, hc), jnp.float32),
            jax.ShapeDtypeStruct((N, co), jnp.float32),
        ),
    )(x, wl, bl.reshape(1, hc), wr, br.reshape(1, hc), wres, bres.reshape(1, co))


def _edge_body(h, c, w_row, src_hbm, dst_hbm, ea_hbm, xl_hbm, xr_hbm, wab_hbm,
               z_hbm, out_hbm, src_v, dst_v, ea_v, xl_b, xr_b, sval, wab_v,
               acc, sem0, sem1):
    hc = h * c
    cid = lax.axis_index("c")
    sid = lax.axis_index("s")

    pltpu.sync_copy(wab_hbm, wab_v)

    rows_per_tile = NPAD // NSUB
    zbase = sid * rows_per_tile
    pltpu.sync_copy(z_hbm.at[pl.ds(zbase, rows_per_tile)],
                    acc.at[pl.ds(zbase, rows_per_tile)])
    plsc.subcore_barrier()

    bbase = (cid * NSUB + sid) * NBLK
    lane_iota = lax.iota(jnp.int32, LANES)

    def chunk(ck, _):
        cbase = pl.multiple_of(bbase + ck * G, G)
        pltpu.sync_copy(src_hbm.at[pl.ds(cbase, G)], src_v)
        pltpu.sync_copy(dst_hbm.at[pl.ds(cbase, G)], dst_v)
        pltpu.sync_copy(ea_hbm.at[pl.ds(cbase, G)], ea_v)
        lax.fori_loop(0, G, block, None)
        return _

    def block(b, _):
        g0 = pltpu.async_copy(xl_hbm.at[src_v.at[b]], xl_b, sem0)
        g1 = pltpu.async_copy(xr_hbm.at[dst_v.at[b]], xr_b, sem1)
        g0.wait()
        g1.wait()

        def subblock(sb, _):
            rows = sb * LANES + lane_iota
            eav = ea_v[b, pl.ds(sb * LANES, LANES)]
            exs = []
            U = 8
            for hh in range(h):
                def jbody(jv, a):
                    for u in range(U):
                        j = hh * c + jv * U + u
                        cj = jnp.full((LANES,), j, jnp.int32)
                        xlv = plsc.load_gather(xl_b, [rows, cj])
                        xrv = plsc.load_gather(xr_b, [rows, cj])
                        wspl = wab_v[0, j, :]
                        aspl = wab_v[1, j, :]
                        t = xlv + xrv + eav * wspl
                        t = jnp.maximum(t, 0.2 * t)
                        a = a + t * aspl
                    return a
                alpha = lax.fori_loop(0, c // U, jbody,
                                      jnp.zeros((LANES,), jnp.float32))
                exs.append(jnp.exp(alpha))
            for hh in range(h):
                exv = exs[hh]
                def j2(jv, c2):
                    for u in range(U):
                        j = hh * c + jv * U + u
                        cj = jnp.full((LANES,), j, jnp.int32)
                        xlv = plsc.load_gather(xl_b, [rows, cj])
                        plsc.store_scatter(sval, [rows, cj], xlv * exv)
                    return c2
                lax.fori_loop(0, c // U, j2, None)
                ch = jnp.full((LANES,), hc + hh, jnp.int32)
                plsc.store_scatter(sval, [rows, ch], exv)
            return _

        lax.fori_loop(0, BLK // LANES, subblock, None)
        pltpu.sync_copy(sval, acc.at[dst_v.at[b]], add=True)
        return _

    lax.fori_loop(0, NBLK // G, chunk, None)
    plsc.subcore_barrier()
    pltpu.sync_copy(acc.at[pl.ds(zbase, rows_per_tile)],
                    out_hbm.at[pl.ds(cid * NPAD + zbase, rows_per_tile)])


def _edge_phase(h, c, w_row, srcp, dstp, eap, xl, xr, wab, zrows):
    hc = h * c
    mesh = plsc.VectorSubcoreMesh(core_axis_name="c", subcore_axis_name="s",
                                  num_cores=NCORE, num_subcores=NSUB)
    body = functools.partial(_edge_body, h, c, w_row)
    f = pl.kernel(
        body,
        out_type=jax.ShapeDtypeStruct((NCORE * NPAD, w_row), jnp.float32),
        mesh=mesh,
        compiler_params=pltpu.CompilerParams(needs_layout_passes=False,
                                             use_tc_tiling_on_sc=False),
        scratch_types=[
            pltpu.VMEM((G, BLK), jnp.int32),
            pltpu.VMEM((G, BLK), jnp.int32),
            pltpu.VMEM((G, BLK), jnp.float32),
            pltpu.VMEM((BLK, hc), jnp.float32),
            pltpu.VMEM((BLK, hc), jnp.float32),
            pltpu.VMEM((BLK, w_row), jnp.float32),
            pltpu.VMEM((2, hc, LANES), jnp.float32),
            pltpu.VMEM_SHARED((NPAD, w_row), jnp.float32),
            pltpu.SemaphoreType.DMA,
            pltpu.SemaphoreType.DMA,
        ],
    )
    return f(srcp, dstp, eap, xl, xr, wab, zrows)


def _post_kernel(h, c, part_ref, bias_ref, g_ref, b_ref, xres_ref, o_ref):
    hc = h * c
    p = part_ref[0:N, :] + part_ref[NPAD:NPAD + N, :]
    cols = []
    for hh in range(h):
        num = p[:, hh * c:(hh + 1) * c]
        den = p[:, hc + hh:hc + hh + 1]
        cols.append(num / (den + 1e-16))
    y = cols[0] if h == 1 else jnp.concatenate(cols, axis=1)
    y = y + bias_ref[...]
    y = jnp.where(y > 0, y, jnp.exp(y) - 1.0)
    mu = jnp.mean(y, axis=-1, keepdims=True)
    var = jnp.mean((y - mu) ** 2, axis=-1, keepdims=True)
    y = (y - mu) / jnp.sqrt(var + 1e-5) * g_ref[...] + b_ref[...]
    o_ref[...] = y + xres_ref[...]


def _post(h, c, part, bias, ln_g, ln_b, xres):
    hc = h * c
    co = hc if h > 1 else c
    kfn = functools.partial(_post_kernel, h, c)
    return pl.pallas_call(
        kfn,
        compiler_params=pltpu.CompilerParams(vmem_limit_bytes=100 * 1024 * 1024),
        out_shape=jax.ShapeDtypeStruct((N, co), jnp.float32),
    )(part, bias.reshape(1, co), ln_g.reshape(1, co), ln_b.reshape(1, co), xres)


def kernel(node_emb, edge_attr, params, edge_index):
    src = edge_index[0].astype(jnp.int32)
    dst = edge_index[1].astype(jnp.int32)
    ea = edge_attr[:, 0].astype(jnp.float32)
    pad = EP - E
    srcp = jnp.concatenate([src, jnp.zeros((pad,), jnp.int32)]).reshape(EP // BLK, BLK)
    dstp = jnp.concatenate([dst, jnp.full((pad,), N, jnp.int32)]).reshape(EP // BLK, BLK)
    eap = jnp.concatenate([ea, jnp.zeros((pad,), jnp.float32)]).reshape(EP // BLK, BLK)

    x = node_emb
    cfgs = [(4, 32, 144), (4, 32, 144), (1, 32, 48)]
    for i, (h, c, w_row) in enumerate(cfgs):
        hc = h * c
        P = params
        if i == 2:
            xl, xr, xres = _dense_proj3(x, P['Wl%d' % i], P['bl%d' % i],
                                        P['Wr%d' % i], P['br%d' % i],
                                        P['Wres2'], P['bres2'])
        else:
            xl, xr = _dense_proj(x, P['Wl%d' % i], P['bl%d' % i],
                                 P['Wr%d' % i], P['br%d' % i])
            xres = x
        wa = jnp.stack([P['We%d' % i].reshape(hc), P['att%d' % i].reshape(hc)])
        wab = jnp.broadcast_to(wa[:, :, None], (2, hc, LANES)).astype(jnp.float32)
        zrows = jnp.zeros((NPAD, w_row), jnp.float32)
        part = _edge_phase(h, c, w_row, srcp, dstp, eap, xl, xr, wab, zrows)
        x = _post(h, c, part, P['bias%d' % i], P['ln_g%d' % i], P['ln_b%d' % i], xres)
    return x


# speedup vs baseline: 9.1569x; 1.0214x over previous
import functools

import jax
import jax.numpy as jnp
from jax import lax
from jax.experimental import pallas as pl
from jax.experimental.pallas import tpu as pltpu
from jax.experimental.pallas import tpu_sc as plsc

N = 10000
E = 320000
D = 128

NCORE = 2
NSUB = 16
BLK = 64
NPAD = 10240
LANES = 16

NBLK = 160
G = 8
EPT = NBLK * BLK
EP = NCORE * NSUB * EPT


def _dense_proj_kernel(x_ref, wl_ref, bl_ref, wr_ref, br_ref, xl_ref, xr_ref):
    x = x_ref[...]
    xl_ref[...] = jnp.dot(x, wl_ref[...], preferred_element_type=jnp.float32) + bl_ref[...]
    xr_ref[...] = jnp.dot(x, wr_ref[...], preferred_element_type=jnp.float32) + br_ref[...]


def _dense_proj(x, wl, bl, wr, br):
    hc = wl.shape[1]
    return pl.pallas_call(
        _dense_proj_kernel,
        compiler_params=pltpu.CompilerParams(vmem_limit_bytes=100 * 1024 * 1024),
        out_shape=(
            jax.ShapeDtypeStruct((N, hc), jnp.float32),
            jax.ShapeDtypeStruct((N, hc), jnp.float32),
        ),
    )(x, wl, bl.reshape(1, hc), wr, br.reshape(1, hc))


def _dense_proj3_kernel(x_ref, wl_ref, bl_ref, wr_ref, br_ref, wres_ref, bres_ref,
                        xl_ref, xr_ref, xres_ref):
    x = x_ref[...]
    xl_ref[...] = jnp.dot(x, wl_ref[...], preferred_element_type=jnp.float32) + bl_ref[...]
    xr_ref[...] = jnp.dot(x, wr_ref[...], preferred_element_type=jnp.float32) + br_ref[...]
    xres_ref[...] = jnp.dot(x, wres_ref[...], preferred_element_type=jnp.float32) + bres_ref[...]


def _dense_proj3(x, wl, bl, wr, br, wres, bres):
    hc = wl.shape[1]
    co = wres.shape[1]
    return pl.pallas_call(
        _dense_proj3_kernel,
        compiler_params=pltpu.CompilerParams(vmem_limit_bytes=100 * 1024 * 1024),
        out_shape=(
            jax.ShapeDtypeStruct((N, hc), jnp.float32),
            jax.ShapeDtypeStruct((N, hc), jnp.float32),
            jax.ShapeDtypeStruct((N, co), jnp.float32),
        ),
    )(x, wl, bl.reshape(1, hc), wr, br.reshape(1, hc), wres, bres.reshape(1, co))


def _edge_body(h, c, w_row, src_hbm, dst_hbm, ea_hbm, xl_hbm, xr_hbm, wab_hbm,
               z_hbm, out_hbm, src_v, dst_v, ea_v, xl_b, xr_b, sval, wab_v,
               acc, sem0, sem1):
    hc = h * c
    cid = lax.axis_index("c")
    sid = lax.axis_index("s")

    pltpu.sync_copy(wab_hbm, wab_v)

    rows_per_tile = NPAD // NSUB
    zbase = sid * rows_per_tile
    pltpu.sync_copy(z_hbm.at[pl.ds(zbase, rows_per_tile)],
                    acc.at[pl.ds(zbase, rows_per_tile)])
    plsc.subcore_barrier()

    bbase = (cid * NSUB + sid) * NBLK
    lane_iota = lax.iota(jnp.int32, LANES)

    def chunk(ck, _):
        cbase = pl.multiple_of(bbase + ck * G, G)
        pltpu.sync_copy(src_hbm.at[pl.ds(cbase, G)], src_v)
        pltpu.sync_copy(dst_hbm.at[pl.ds(cbase, G)], dst_v)
        pltpu.sync_copy(ea_hbm.at[pl.ds(cbase, G)], ea_v)
        lax.fori_loop(0, G, block, None)
        return _

    def block(b, _):
        g0 = pltpu.async_copy(xl_hbm.at[src_v.at[b]], xl_b, sem0)
        g1 = pltpu.async_copy(xr_hbm.at[dst_v.at[b]], xr_b, sem1)
        g0.wait()
        g1.wait()

        def subblock(sb, _):
            rows = sb * LANES + lane_iota
            eav = ea_v[b, pl.ds(sb * LANES, LANES)]
            exs = []
            U = 8
            zero16 = jnp.zeros((LANES,), jnp.float32)
            for hh in range(h):
                def jbody(jv, accs):
                    accs = list(accs)
                    for u in range(U):
                        j = hh * c + jv * U + u
                        cj = jnp.full((LANES,), j, jnp.int32)
                        xlv = plsc.load_gather(xl_b, [rows, cj])
                        xrv = plsc.load_gather(xr_b, [rows, cj])
                        wspl = wab_v[0, j, :]
                        aspl = wab_v[1, j, :]
                        t = xlv + xrv + eav * wspl
                        t = jnp.maximum(t, 0.2 * t)
                        accs[u % 4] = accs[u % 4] + t * aspl
                    return tuple(accs)
                a0, a1, a2, a3 = lax.fori_loop(0, c // U, jbody,
                                               (zero16, zero16, zero16, zero16))
                exs.append(jnp.exp((a0 + a1) + (a2 + a3)))
            for hh in range(h):
                exv = exs[hh]
                def j2(jv, c2):
                    for u in range(U):
                        j = hh * c + jv * U + u
                        cj = jnp.full((LANES,), j, jnp.int32)
                        xlv = plsc.load_gather(xl_b, [rows, cj])
                        plsc.store_scatter(sval, [rows, cj], xlv * exv)
                    return c2
                lax.fori_loop(0, c // U, j2, None)
                ch = jnp.full((LANES,), hc + hh, jnp.int32)
                plsc.store_scatter(sval, [rows, ch], exv)
            return _

        lax.fori_loop(0, BLK // LANES, subblock, None)
        pltpu.sync_copy(sval, acc.at[dst_v.at[b]], add=True)
        return _

    lax.fori_loop(0, NBLK // G, chunk, None)
    plsc.subcore_barrier()
    pltpu.sync_copy(acc.at[pl.ds(zbase, rows_per_tile)],
                    out_hbm.at[pl.ds(cid * NPAD + zbase, rows_per_tile)])


def _edge_phase(h, c, w_row, srcp, dstp, eap, xl, xr, wab, zrows):
    hc = h * c
    mesh = plsc.VectorSubcoreMesh(core_axis_name="c", subcore_axis_name="s",
                                  num_cores=NCORE, num_subcores=NSUB)
    body = functools.partial(_edge_body, h, c, w_row)
    f = pl.kernel(
        body,
        out_type=jax.ShapeDtypeStruct((NCORE * NPAD, w_row), jnp.float32),
        mesh=mesh,
        compiler_params=pltpu.CompilerParams(needs_layout_passes=False,
                                             use_tc_tiling_on_sc=False),
        scratch_types=[
            pltpu.VMEM((G, BLK), jnp.int32),
            pltpu.VMEM((G, BLK), jnp.int32),
            pltpu.VMEM((G, BLK), jnp.float32),
            pltpu.VMEM((BLK, hc), jnp.float32),
            pltpu.VMEM((BLK, hc), jnp.float32),
            pltpu.VMEM((BLK, w_row), jnp.float32),
            pltpu.VMEM((2, hc, LANES), jnp.float32),
            pltpu.VMEM_SHARED((NPAD, w_row), jnp.float32),
            pltpu.SemaphoreType.DMA,
            pltpu.SemaphoreType.DMA,
        ],
    )
    return f(srcp, dstp, eap, xl, xr, wab, zrows)


def _post_kernel(h, c, part_ref, bias_ref, g_ref, b_ref, xres_ref, o_ref):
    hc = h * c
    p = part_ref[0:N, :] + part_ref[NPAD:NPAD + N, :]
    cols = []
    for hh in range(h):
        num = p[:, hh * c:(hh + 1) * c]
        den = p[:, hc + hh:hc + hh + 1]
        cols.append(num / (den + 1e-16))
    y = cols[0] if h == 1 else jnp.concatenate(cols, axis=1)
    y = y + bias_ref[...]
    y = jnp.where(y > 0, y, jnp.exp(y) - 1.0)
    mu = jnp.mean(y, axis=-1, keepdims=True)
    var = jnp.mean((y - mu) ** 2, axis=-1, keepdims=True)
    y = (y - mu) / jnp.sqrt(var + 1e-5) * g_ref[...] + b_ref[...]
    o_ref[...] = y + xres_ref[...]


def _post(h, c, part, bias, ln_g, ln_b, xres):
    hc = h * c
    co = hc if h > 1 else c
    kfn = functools.partial(_post_kernel, h, c)
    return pl.pallas_call(
        kfn,
        compiler_params=pltpu.CompilerParams(vmem_limit_bytes=100 * 1024 * 1024),
        out_shape=jax.ShapeDtypeStruct((N, co), jnp.float32),
    )(part, bias.reshape(1, co), ln_g.reshape(1, co), ln_b.reshape(1, co), xres)


def kernel(node_emb, edge_attr, params, edge_index):
    src = edge_index[0].astype(jnp.int32)
    dst = edge_index[1].astype(jnp.int32)
    ea = edge_attr[:, 0].astype(jnp.float32)
    pad = EP - E
    srcp = jnp.concatenate([src, jnp.zeros((pad,), jnp.int32)]).reshape(EP // BLK, BLK)
    dstp = jnp.concatenate([dst, jnp.full((pad,), N, jnp.int32)]).reshape(EP // BLK, BLK)
    eap = jnp.concatenate([ea, jnp.zeros((pad,), jnp.float32)]).reshape(EP // BLK, BLK)

    x = node_emb
    cfgs = [(4, 32, 144), (4, 32, 144), (1, 32, 48)]
    for i, (h, c, w_row) in enumerate(cfgs):
        hc = h * c
        P = params
        if i == 2:
            xl, xr, xres = _dense_proj3(x, P['Wl%d' % i], P['bl%d' % i],
                                        P['Wr%d' % i], P['br%d' % i],
                                        P['Wres2'], P['bres2'])
        else:
            xl, xr = _dense_proj(x, P['Wl%d' % i], P['bl%d' % i],
                                 P['Wr%d' % i], P['br%d' % i])
            xres = x
        wa = jnp.stack([P['We%d' % i].reshape(hc), P['att%d' % i].reshape(hc)])
        wab = jnp.broadcast_to(wa[:, :, None], (2, hc, LANES)).astype(jnp.float32)
        zrows = jnp.zeros((NPAD, w_row), jnp.float32)
        part = _edge_phase(h, c, w_row, srcp, dstp, eap, xl, xr, wab, zrows)
        x = _post(h, c, part, P['bias%d' % i], P['ln_g%d' % i], P['ln_b%d' % i], xres)
    return x


# speedup vs baseline: 10.6899x; 1.1674x over previous
import functools

import jax
import jax.numpy as jnp
from jax import lax
from jax.experimental import pallas as pl
from jax.experimental.pallas import tpu as pltpu
from jax.experimental.pallas import tpu_sc as plsc

N = 10000
E = 320000
D = 128

NCORE = 2
NSUB = 16
BLK = 64
NPAD = 10240
LANES = 16

NBLK = 160
G = 8
EPT = NBLK * BLK
EP = NCORE * NSUB * EPT


def _dense_proj_kernel(x_ref, wl_ref, bl_ref, wr_ref, br_ref, xl_ref, xr_ref):
    x = x_ref[...]
    xl_ref[...] = jnp.dot(x, wl_ref[...], preferred_element_type=jnp.float32) + bl_ref[...]
    xr_ref[...] = jnp.dot(x, wr_ref[...], preferred_element_type=jnp.float32) + br_ref[...]


def _dense_proj(x, wl, bl, wr, br):
    hc = wl.shape[1]
    return pl.pallas_call(
        _dense_proj_kernel,
        compiler_params=pltpu.CompilerParams(vmem_limit_bytes=100 * 1024 * 1024),
        out_shape=(
            jax.ShapeDtypeStruct((N, hc), jnp.float32),
            jax.ShapeDtypeStruct((N, hc), jnp.float32),
        ),
    )(x, wl, bl.reshape(1, hc), wr, br.reshape(1, hc))


def _dense_proj3_kernel(x_ref, wl_ref, bl_ref, wr_ref, br_ref, wres_ref, bres_ref,
                        xl_ref, xr_ref, xres_ref):
    x = x_ref[...]
    xl_ref[...] = jnp.dot(x, wl_ref[...], preferred_element_type=jnp.float32) + bl_ref[...]
    xr_ref[...] = jnp.dot(x, wr_ref[...], preferred_element_type=jnp.float32) + br_ref[...]
    xres_ref[...] = jnp.dot(x, wres_ref[...], preferred_element_type=jnp.float32) + bres_ref[...]


def _dense_proj3(x, wl, bl, wr, br, wres, bres):
    hc = wl.shape[1]
    co = wres.shape[1]
    return pl.pallas_call(
        _dense_proj3_kernel,
        compiler_params=pltpu.CompilerParams(vmem_limit_bytes=100 * 1024 * 1024),
        out_shape=(
            jax.ShapeDtypeStruct((N, hc), jnp.float32),
            jax.ShapeDtypeStruct((N, hc), jnp.float32),
            jax.ShapeDtypeStruct((N, co), jnp.float32),
        ),
    )(x, wl, bl.reshape(1, hc), wr, br.reshape(1, hc), wres, bres.reshape(1, co))


def _edge_body(h, c, w_row, src_hbm, dst_hbm, ea_hbm, xl_hbm, xr_hbm, wab_hbm,
               z_hbm, out_hbm, src_v, dst_v, ea_v, xl_b, xr_b, sval, wab_v,
               acc, sem0, sem1):
    hc = h * c
    cid = lax.axis_index("c")
    sid = lax.axis_index("s")

    pltpu.sync_copy(wab_hbm, wab_v)

    rows_per_tile = NPAD // NSUB
    zbase = sid * rows_per_tile
    pltpu.sync_copy(z_hbm.at[pl.ds(zbase, rows_per_tile)],
                    acc.at[pl.ds(zbase, rows_per_tile)])
    plsc.subcore_barrier()

    bbase = (cid * NSUB + sid) * NBLK
    lane_iota = lax.iota(jnp.int32, LANES)

    def chunk(ck, _):
        cbase = pl.multiple_of(bbase + ck * G, G)
        pltpu.sync_copy(src_hbm.at[pl.ds(cbase, G)], src_v)
        pltpu.sync_copy(dst_hbm.at[pl.ds(cbase, G)], dst_v)
        pltpu.sync_copy(ea_hbm.at[pl.ds(cbase, G)], ea_v)
        lax.fori_loop(0, G, block, None)
        return _

    def block(b, _):
        g0 = pltpu.async_copy(xl_hbm.at[src_v.at[b]], xl_b, sem0)
        g1 = pltpu.async_copy(xr_hbm.at[dst_v.at[b]], xr_b, sem1)
        g0.wait()
        g1.wait()

        def subblock(sb, _):
            rows = sb * LANES + lane_iota
            eav = ea_v[b, pl.ds(sb * LANES, LANES)]
            exs = []
            zero16 = jnp.zeros((LANES,), jnp.float32)
            for hh in range(h):
                @plsc.parallel_loop(hh * c, (hh + 1) * c, unroll=8,
                                    carry=(zero16, zero16))
                def pA(j, accs):
                    a0, a1 = accs
                    cj = jnp.full((LANES,), j, jnp.int32)
                    xlv = plsc.load_gather(xl_b, [rows, cj])
                    xrv = plsc.load_gather(xr_b, [rows, cj])
                    wspl = wab_v[0, j, :]
                    aspl = wab_v[1, j, :]
                    t = xlv + xrv + eav * wspl
                    t = jnp.maximum(t, 0.2 * t)
                    return (a1, a0 + t * aspl)
                a0, a1 = pA
                exs.append(jnp.exp(a0 + a1))
            for hh in range(h):
                exv = exs[hh]
                @plsc.parallel_loop(hh * c, (hh + 1) * c, unroll=8)
                def pB(j):
                    cj = jnp.full((LANES,), j, jnp.int32)
                    xlv = plsc.load_gather(xl_b, [rows, cj])
                    plsc.store_scatter(sval, [rows, cj], xlv * exv)
                ch = jnp.full((LANES,), hc + hh, jnp.int32)
                plsc.store_scatter(sval, [rows, ch], exv)
            return _

        lax.fori_loop(0, BLK // LANES, subblock, None)
        pltpu.sync_copy(sval, acc.at[dst_v.at[b]], add=True)
        return _

    lax.fori_loop(0, NBLK // G, chunk, None)
    plsc.subcore_barrier()
    pltpu.sync_copy(acc.at[pl.ds(zbase, rows_per_tile)],
                    out_hbm.at[pl.ds(cid * NPAD + zbase, rows_per_tile)])


def _edge_phase(h, c, w_row, srcp, dstp, eap, xl, xr, wab, zrows):
    hc = h * c
    mesh = plsc.VectorSubcoreMesh(core_axis_name="c", subcore_axis_name="s",
                                  num_cores=NCORE, num_subcores=NSUB)
    body = functools.partial(_edge_body, h, c, w_row)
    f = pl.kernel(
        body,
        out_type=jax.ShapeDtypeStruct((NCORE * NPAD, w_row), jnp.float32),
        mesh=mesh,
        compiler_params=pltpu.CompilerParams(needs_layout_passes=False,
                                             use_tc_tiling_on_sc=False),
        scratch_types=[
            pltpu.VMEM((G, BLK), jnp.int32),
            pltpu.VMEM((G, BLK), jnp.int32),
            pltpu.VMEM((G, BLK), jnp.float32),
            pltpu.VMEM((BLK, hc), jnp.float32),
            pltpu.VMEM((BLK, hc), jnp.float32),
            pltpu.VMEM((BLK, w_row), jnp.float32),
            pltpu.VMEM((2, hc, LANES), jnp.float32),
            pltpu.VMEM_SHARED((NPAD, w_row), jnp.float32),
            pltpu.SemaphoreType.DMA,
            pltpu.SemaphoreType.DMA,
        ],
    )
    return f(srcp, dstp, eap, xl, xr, wab, zrows)


def _post_kernel(h, c, part_ref, bias_ref, g_ref, b_ref, xres_ref, o_ref):
    hc = h * c
    p = part_ref[0:N, :] + part_ref[NPAD:NPAD + N, :]
    cols = []
    for hh in range(h):
        num = p[:, hh * c:(hh + 1) * c]
        den = p[:, hc + hh:hc + hh + 1]
        cols.append(num / (den + 1e-16))
    y = cols[0] if h == 1 else jnp.concatenate(cols, axis=1)
    y = y + bias_ref[...]
    y = jnp.where(y > 0, y, jnp.exp(y) - 1.0)
    mu = jnp.mean(y, axis=-1, keepdims=True)
    var = jnp.mean((y - mu) ** 2, axis=-1, keepdims=True)
    y = (y - mu) / jnp.sqrt(var + 1e-5) * g_ref[...] + b_ref[...]
    o_ref[...] = y + xres_ref[...]


def _post(h, c, part, bias, ln_g, ln_b, xres):
    hc = h * c
    co = hc if h > 1 else c
    kfn = functools.partial(_post_kernel, h, c)
    return pl.pallas_call(
        kfn,
        compiler_params=pltpu.CompilerParams(vmem_limit_bytes=100 * 1024 * 1024),
        out_shape=jax.ShapeDtypeStruct((N, co), jnp.float32),
    )(part, bias.reshape(1, co), ln_g.reshape(1, co), ln_b.reshape(1, co), xres)


def kernel(node_emb, edge_attr, params, edge_index):
    src = edge_index[0].astype(jnp.int32)
    dst = edge_index[1].astype(jnp.int32)
    ea = edge_attr[:, 0].astype(jnp.float32)
    pad = EP - E
    srcp = jnp.concatenate([src, jnp.zeros((pad,), jnp.int32)]).reshape(EP // BLK, BLK)
    dstp = jnp.concatenate([dst, jnp.full((pad,), N, jnp.int32)]).reshape(EP // BLK, BLK)
    eap = jnp.concatenate([ea, jnp.zeros((pad,), jnp.float32)]).reshape(EP // BLK, BLK)

    x = node_emb
    cfgs = [(4, 32, 144), (4, 32, 144), (1, 32, 48)]
    for i, (h, c, w_row) in enumerate(cfgs):
        hc = h * c
        P = params
        if i == 2:
            xl, xr, xres = _dense_proj3(x, P['Wl%d' % i], P['bl%d' % i],
                                        P['Wr%d' % i], P['br%d' % i],
                                        P['Wres2'], P['bres2'])
        else:
            xl, xr = _dense_proj(x, P['Wl%d' % i], P['bl%d' % i],
                                 P['Wr%d' % i], P['br%d' % i])
            xres = x
        wa = jnp.stack([P['We%d' % i].reshape(hc), P['att%d' % i].reshape(hc)])
        wab = jnp.broadcast_to(wa[:, :, None], (2, hc, LANES)).astype(jnp.float32)
        zrows = jnp.zeros((NPAD, w_row), jnp.float32)
        part = _edge_phase(h, c, w_row, srcp, dstp, eap, xl, xr, wab, zrows)
        x = _post(h, c, part, P['bias%d' % i], P['ln_g%d' % i], P['ln_b%d' % i], xres)
    return x
